# baseline (speedup 1.0000x reference)
import jax
import jax.numpy as jnp
from jax.experimental import pallas as pl
from jax.experimental.pallas import tpu as pltpu

_D = 32
_K = 8192
_R = 4
_TT = 256
_CHUNKS = (8192, 2048, 2048, 4096)


def _refine_kernel(cb_ref, w1_ref, b1_ref, w2_ref, b2_ref,
                   cb2_ref, cb2b_ref, cn_ref):
    for i in range(_R):
        cb = cb_ref[i].astype(jnp.bfloat16)
        w1 = w1_ref[i].astype(jnp.bfloat16)
        h = jax.lax.dot_general(cb, w1, (((1,), (1,)), ((), ())),
                                preferred_element_type=jnp.float32)
        h = jnp.maximum(h + b1_ref[i], 0.0).astype(jnp.bfloat16)
        w2 = w2_ref[i].astype(jnp.bfloat16)
        cb2 = jax.lax.dot_general(h, w2, (((1,), (1,)), ((), ())),
                                  preferred_element_type=jnp.float32)
        cb2 = cb2 + b2_ref[i]
        cb2_ref[i] = cb2
        cb2b_ref[i] = cb2.astype(jnp.bfloat16)
        cn_ref[i] = jnp.sum(cb2 * cb2, axis=1)[None, :]


def _vq_kernel(z_ref, cb2_ref, cb2b_ref, cn_ref, out_ref):
    r = z_ref[...]
    acc = jnp.zeros((_TT, _D), jnp.float32)
    for i in range(_R):
        cb = cb2_ref[i]
        s = jax.lax.dot_general(r.astype(jnp.bfloat16), cb2b_ref[i],
                                (((1,), (1,)), ((), ())),
                                preferred_element_type=jnp.float32)
        rr = jnp.sum(r * r, axis=1, keepdims=True)
        dist = rr - 2.0 * s + cn_ref[i]
        ch = _CHUNKS[i]
        nch = _K // ch
        if nch == 1:
            idx = jnp.argmin(dist, axis=1).astype(jnp.int32)[:, None]
        else:
            carry_v = jnp.full((_TT, 1), jnp.inf, jnp.float32)
            carry_i = jnp.zeros((_TT, 1), jnp.int32)
            for b in range(nch):
                dblk = dist[:, b * ch:(b + 1) * ch]
                newv = jnp.min(dblk, axis=1, keepdims=True)
                newi = (jnp.argmin(dblk, axis=1).astype(jnp.int32)[:, None]
                        + b * ch)
                take = newv < carry_v
                carry_i = jnp.where(take, newi, carry_i)
                carry_v = jnp.where(take, newv, carry_v)
                carry_v = carry_v.astype(jnp.bfloat16).astype(jnp.float32)
            idx = carry_i
        onehot = (jax.lax.broadcasted_iota(jnp.int32, (_TT, _K), 1)
                  == idx).astype(jnp.float32)
        q = jax.lax.dot_general(onehot, cb, (((1,), (0,)), ((), ())),
                                preferred_element_type=jnp.float32,
                                precision=jax.lax.Precision.HIGHEST)
        acc = acc + q
        r = r - q
    out_ref[...] = acc


def kernel(z, codebooks, W1, b1, W2, b2):
    B, N, D = z.shape
    zf = z.reshape(-1, D)
    cb2, cb2b, cn = pl.pallas_call(
        _refine_kernel,
        out_shape=[jax.ShapeDtypeStruct((_R, _K, _D), jnp.float32),
                   jax.ShapeDtypeStruct((_R, _K, _D), jnp.bfloat16),
                   jax.ShapeDtypeStruct((_R, 1, _K), jnp.float32)],
    )(codebooks, W1, b1.reshape(_R, 1, _D), W2, b2.reshape(_R, 1, _D))
    T = zf.shape[0]
    out = pl.pallas_call(
        _vq_kernel,
        grid=(T // _TT,),
        in_specs=[
            pl.BlockSpec((_TT, _D), lambda i: (i, 0)),
            pl.BlockSpec((_R, _K, _D), lambda i: (0, 0, 0)),
            pl.BlockSpec((_R, _K, _D), lambda i: (0, 0, 0)),
            pl.BlockSpec((_R, 1, _K), lambda i: (0, 0, 0)),
        ],
        out_specs=pl.BlockSpec((_TT, _D), lambda i: (i, 0)),
        out_shape=jax.ShapeDtypeStruct((T, _D), jnp.float32),
        compiler_params=pltpu.CompilerParams(
            dimension_semantics=("parallel",)),
    )(zf, cb2, cb2b, cn)
    return out.reshape(B, N, D)


# speedup vs baseline: 2.7741x; 2.7741x over previous
import jax
import jax.numpy as jnp
from jax.experimental import pallas as pl
from jax.experimental.pallas import tpu as pltpu

_D = 32
_K = 8192
_R = 4
_TT = 256
_CHUNKS = (8192, 2048, 2048, 4096)
_BF = jnp.bfloat16


def _refine_kernel(cb_ref, w1_ref, b1_ref, w2_ref, b2_ref,
                   cb2b_ref, mid_ref, lo_ref, cn_ref):
    cb = cb_ref[0].astype(_BF)
    w1 = w1_ref[0].astype(_BF)
    h = jax.lax.dot_general(cb, w1, (((1,), (1,)), ((), ())),
                            preferred_element_type=jnp.float32)
    h = jnp.maximum(h + b1_ref[0], 0.0).astype(_BF)
    w2 = w2_ref[0].astype(_BF)
    cb2 = jax.lax.dot_general(h, w2, (((1,), (1,)), ((), ())),
                              preferred_element_type=jnp.float32)
    cb2 = cb2 + b2_ref[0]
    hi = cb2.astype(_BF)
    rem = cb2 - hi.astype(jnp.float32)
    mid = rem.astype(_BF)
    lo = (rem - mid.astype(jnp.float32)).astype(_BF)
    cb2b_ref[0] = hi
    mid_ref[0] = mid
    lo_ref[0] = lo
    cn_ref[0] = jnp.sum(cb2 * cb2, axis=1)[None, :]


def _vq_kernel(z_ref, cb2b_ref, mid_ref, lo_ref, cn_ref, out_ref):
    r = z_ref[...]
    acc = jnp.zeros((_TT, _D), jnp.float32)
    for i in range(_R):
        s = jax.lax.dot_general(r.astype(_BF), cb2b_ref[i],
                                (((1,), (1,)), ((), ())),
                                preferred_element_type=jnp.float32)
        rr = jnp.sum(r * r, axis=1, keepdims=True)
        dist = rr - 2.0 * s + cn_ref[i]
        ch = _CHUNKS[i]
        nch = _K // ch
        if nch == 1:
            idx = jnp.argmin(dist, axis=1).astype(jnp.int32)[:, None]
        else:
            carry_v = jnp.full((_TT, 1), jnp.inf, jnp.float32)
            carry_i = jnp.zeros((_TT, 1), jnp.int32)
            for b in range(nch):
                dblk = dist[:, b * ch:(b + 1) * ch]
                newv = jnp.min(dblk, axis=1, keepdims=True)
                newi = (jnp.argmin(dblk, axis=1).astype(jnp.int32)[:, None]
                        + b * ch)
                take = newv < carry_v
                carry_i = jnp.where(take, newi, carry_i)
                carry_v = jnp.where(take, newv, carry_v)
                carry_v = carry_v.astype(_BF).astype(jnp.float32)
            idx = carry_i
        onehot = (jax.lax.broadcasted_iota(jnp.int32, (_TT, _K), 1)
                  == idx).astype(_BF)
        q = jnp.zeros((_TT, _D), jnp.float32)
        for part_ref in (cb2b_ref, mid_ref, lo_ref):
            q = q + jax.lax.dot_general(onehot, part_ref[i],
                                        (((1,), (0,)), ((), ())),
                                        preferred_element_type=jnp.float32)
        acc = acc + q
        r = r - q
    out_ref[...] = acc


def kernel(z, codebooks, W1, b1, W2, b2):
    B, N, D = z.shape
    zf = z.reshape(-1, D)
    _cbspec = pl.BlockSpec((1, _K, _D), lambda i: (i, 0, 0))
    cb2b, mid, lo, cn = pl.pallas_call(
        _refine_kernel,
        grid=(_R,),
        in_specs=[
            _cbspec,
            pl.BlockSpec((1, _D, _D), lambda i: (i, 0, 0)),
            pl.BlockSpec((1, 1, _D), lambda i: (i, 0, 0)),
            pl.BlockSpec((1, _D, _D), lambda i: (i, 0, 0)),
            pl.BlockSpec((1, 1, _D), lambda i: (i, 0, 0)),
        ],
        out_specs=[_cbspec, _cbspec, _cbspec,
                   pl.BlockSpec((1, 1, _K), lambda i: (i, 0, 0))],
        out_shape=[jax.ShapeDtypeStruct((_R, _K, _D), _BF),
                   jax.ShapeDtypeStruct((_R, _K, _D), _BF),
                   jax.ShapeDtypeStruct((_R, _K, _D), _BF),
                   jax.ShapeDtypeStruct((_R, 1, _K), jnp.float32)],
        compiler_params=pltpu.CompilerParams(
            dimension_semantics=("arbitrary",)),
    )(codebooks, W1, b1.reshape(_R, 1, _D), W2, b2.reshape(_R, 1, _D))
    T = zf.shape[0]
    out = pl.pallas_call(
        _vq_kernel,
        grid=(T // _TT,),
        in_specs=[
            pl.BlockSpec((_TT, _D), lambda i: (i, 0)),
            pl.BlockSpec((_R, _K, _D), lambda i: (0, 0, 0)),
            pl.BlockSpec((_R, _K, _D), lambda i: (0, 0, 0)),
            pl.BlockSpec((_R, _K, _D), lambda i: (0, 0, 0)),
            pl.BlockSpec((_R, 1, _K), lambda i: (0, 0, 0)),
        ],
        out_specs=pl.BlockSpec((_TT, _D), lambda i: (i, 0)),
        out_shape=jax.ShapeDtypeStruct((T, _D), jnp.float32),
        compiler_params=pltpu.CompilerParams(
            dimension_semantics=("parallel",)),
    )(zf, cb2b, mid, lo, cn)
    return out.reshape(B, N, D)


# speedup vs baseline: 2.9175x; 1.0517x over previous
import jax
import jax.numpy as jnp
from jax.experimental import pallas as pl
from jax.experimental.pallas import tpu as pltpu

_D = 32
_K = 8192
_R = 4
_TT = 256
_CHUNKS = (8192, 2048, 2048, 4096)
_BF = jnp.bfloat16


def _refine_kernel(cb_ref, w1_ref, b1_ref, w2_ref, b2_ref,
                   cb2b_ref, cb2x2_ref, mid_ref, lo_ref, cn_ref):
    cb = cb_ref[0].astype(_BF)
    w1 = w1_ref[0].astype(_BF)
    h = jax.lax.dot_general(cb, w1, (((1,), (1,)), ((), ())),
                            preferred_element_type=jnp.float32)
    h = jnp.maximum(h + b1_ref[0], 0.0).astype(_BF)
    w2 = w2_ref[0].astype(_BF)
    cb2 = jax.lax.dot_general(h, w2, (((1,), (1,)), ((), ())),
                              preferred_element_type=jnp.float32)
    cb2 = cb2 + b2_ref[0]
    hi = cb2.astype(_BF)
    rem = cb2 - hi.astype(jnp.float32)
    mid = rem.astype(_BF)
    lo = (rem - mid.astype(jnp.float32)).astype(_BF)
    cb2b_ref[0] = hi
    cb2x2_ref[0] = (hi.astype(jnp.float32) * 2.0).astype(_BF)
    mid_ref[0] = mid
    lo_ref[0] = lo
    cn_ref[0] = jnp.sum(cb2 * cb2, axis=1)[None, :]


def _vq_kernel(z_ref, cb2b_ref, cb2x2_ref, mid_ref, lo_ref, cn_ref, out_ref):
    r = z_ref[...]
    acc = jnp.zeros((_TT, _D), jnp.float32)
    for i in range(_R):
        s2 = jax.lax.dot_general(r.astype(_BF), cb2x2_ref[i],
                                 (((1,), (1,)), ((), ())),
                                 preferred_element_type=jnp.float32)
        rr = jnp.sum(r * r, axis=1, keepdims=True)
        dist = rr - s2 + cn_ref[i]
        ch = _CHUNKS[i]
        nch = _K // ch
        if nch == 1:
            idx = jnp.argmin(dist, axis=1).astype(jnp.int32)[:, None]
        else:
            carry_v = jnp.full((_TT, 1), jnp.inf, jnp.float32)
            carry_i = jnp.zeros((_TT, 1), jnp.int32)
            for b in range(nch):
                dblk = dist[:, b * ch:(b + 1) * ch]
                newv = jnp.min(dblk, axis=1, keepdims=True)
                newi = (jnp.argmin(dblk, axis=1).astype(jnp.int32)[:, None]
                        + b * ch)
                take = newv < carry_v
                carry_i = jnp.where(take, newi, carry_i)
                carry_v = jnp.where(take, newv, carry_v)
                carry_v = carry_v.astype(_BF).astype(jnp.float32)
            idx = carry_i
        onehot = (jax.lax.broadcasted_iota(jnp.int32, (_TT, _K), 1)
                  == idx).astype(_BF)
        q = jnp.zeros((_TT, _D), jnp.float32)
        for part_ref in (cb2b_ref, mid_ref, lo_ref):
            q = q + jax.lax.dot_general(onehot, part_ref[i],
                                        (((1,), (0,)), ((), ())),
                                        preferred_element_type=jnp.float32)
        acc = acc + q
        r = r - q
    out_ref[...] = acc


def kernel(z, codebooks, W1, b1, W2, b2):
    B, N, D = z.shape
    zf = z.reshape(-1, D)
    _cbspec = pl.BlockSpec((1, _K, _D), lambda i: (i, 0, 0))
    cb2b, cb2x2, mid, lo, cn = pl.pallas_call(
        _refine_kernel,
        grid=(_R,),
        in_specs=[
            _cbspec,
            pl.BlockSpec((1, _D, _D), lambda i: (i, 0, 0)),
            pl.BlockSpec((1, 1, _D), lambda i: (i, 0, 0)),
            pl.BlockSpec((1, _D, _D), lambda i: (i, 0, 0)),
            pl.BlockSpec((1, 1, _D), lambda i: (i, 0, 0)),
        ],
        out_specs=[_cbspec, _cbspec, _cbspec, _cbspec,
                   pl.BlockSpec((1, 1, _K), lambda i: (i, 0, 0))],
        out_shape=[jax.ShapeDtypeStruct((_R, _K, _D), _BF),
                   jax.ShapeDtypeStruct((_R, _K, _D), _BF),
                   jax.ShapeDtypeStruct((_R, _K, _D), _BF),
                   jax.ShapeDtypeStruct((_R, _K, _D), _BF),
                   jax.ShapeDtypeStruct((_R, 1, _K), jnp.float32)],
        compiler_params=pltpu.CompilerParams(
            dimension_semantics=("arbitrary",)),
    )(codebooks, W1, b1.reshape(_R, 1, _D), W2, b2.reshape(_R, 1, _D))
    T = zf.shape[0]
    out = pl.pallas_call(
        _vq_kernel,
        grid=(T // _TT,),
        in_specs=[
            pl.BlockSpec((_TT, _D), lambda i: (i, 0)),
            pl.BlockSpec((_R, _K, _D), lambda i: (0, 0, 0)),
            pl.BlockSpec((_R, _K, _D), lambda i: (0, 0, 0)),
            pl.BlockSpec((_R, _K, _D), lambda i: (0, 0, 0)),
            pl.BlockSpec((_R, _K, _D), lambda i: (0, 0, 0)),
            pl.BlockSpec((_R, 1, _K), lambda i: (0, 0, 0)),
        ],
        out_specs=pl.BlockSpec((_TT, _D), lambda i: (i, 0)),
        out_shape=jax.ShapeDtypeStruct((T, _D), jnp.float32),
        compiler_params=pltpu.CompilerParams(
            dimension_semantics=("parallel",)),
    )(zf, cb2b, cb2x2, mid, lo, cn)
    return out.reshape(B, N, D)
